# baseline (speedup 1.0000x reference)
import functools

import jax
import jax.numpy as jnp
from jax import lax
from jax.experimental import pallas as pl
from jax.experimental.pallas import tpu as pltpu
from jax.experimental.pallas import tpu_sc as plsc

N_NODES = 10000
N_EDGES = 320000
D = 128
D_EDGE = 16

NC, NS = 2, 16
NW = NC * NS
CHUNK = 160
NSLOT = 2
BATCH_A = 88
BATCH_B = 40
BATCHES = BATCH_A + BATCH_B
PER_W_MAX = CHUNK * BATCH_A
N_PAD = CHUNK * BATCHES * NS
IDX_PAD = N_PAD + (BATCH_A - BATCH_B) * CHUNK


def _project_body(x_ref, w1_ref, w2_ref, p1_ref, p2_ref):
    x = x_ref[...]
    p1_ref[...] = jnp.dot(x, w1_ref[...], preferred_element_type=jnp.float32)
    p2_ref[...] = jnp.dot(x, w2_ref[...], preferred_element_type=jnp.float32)


def _project(node_attr, w1, w2):
    blk = 1000
    grid = N_NODES // blk
    return pl.pallas_call(
        _project_body,
        grid=(grid,),
        in_specs=[
            pl.BlockSpec((blk, D), lambda i: (i, 0)),
            pl.BlockSpec((D, D), lambda i: (0, 0)),
            pl.BlockSpec((D, D), lambda i: (0, 0)),
        ],
        out_specs=[
            pl.BlockSpec((blk, D), lambda i: (i, 0)),
            pl.BlockSpec((blk, D), lambda i: (i, 0)),
        ],
        out_shape=[
            jax.ShapeDtypeStruct((N_NODES, D), jnp.float32),
            jax.ShapeDtypeStruct((N_NODES, D), jnp.float32),
        ],
    )(node_attr, w1, w2)


def _gather_sum_body(p1_hbm, p2_hbm, sidx_hbm, ridx_hbm, g_hbm,
                     idxs_v, idxr_v,
                     b1_0, b2_0, b1_1, b2_1, gs0, gs1, ws0, ws1):
    cid = lax.axis_index("c")
    sid = lax.axis_index("s")
    nb = jnp.where(cid == 0, BATCH_A, BATCH_B)
    e0 = jnp.where(cid == 0,
                   sid * (BATCH_A * CHUNK),
                   NS * (BATCH_A * CHUNK) + sid * (BATCH_B * CHUNK))
    e0 = pl.multiple_of(e0, 8)
    pltpu.sync_copy(sidx_hbm.at[pl.ds(e0, PER_W_MAX)], idxs_v)
    pltpu.sync_copy(ridx_hbm.at[pl.ds(e0, PER_W_MAX)], idxr_v)

    slots = ((b1_0, b2_0, gs0, ws0), (b1_1, b2_1, gs1, ws1))

    def issue_gather(b, g1, g2, gs):
        off = pl.multiple_of(b * CHUNK, 8)
        pltpu.async_copy(p1_hbm.at[idxs_v.at[pl.ds(off, CHUNK)]], g1, gs)
        pltpu.async_copy(p2_hbm.at[idxr_v.at[pl.ds(off, CHUNK)]], g2, gs)

    for k in range(NSLOT):
        issue_gather(k, slots[k][0], slots[k][1], slots[k][2])

    def outer(it, carry):
        for k in range(NSLOT):
            g1, g2, gs, ws = slots[k]
            kp = (k + NSLOT - 1) % NSLOT
            g1p, g2p, gsp, wsp = slots[kp]
            bi = it * NSLOT + k

            @pl.when(jnp.logical_and(bi >= 1, bi + NSLOT - 1 < nb))
            def _():
                pltpu.make_async_copy(
                    g1p, g_hbm.at[pl.ds(0, CHUNK)], wsp).wait()
                issue_gather(bi + NSLOT - 1, g1p, g2p, gsp)

            pltpu.make_async_copy(p1_hbm.at[pl.ds(0, CHUNK)], g1, gs).wait()
            pltpu.make_async_copy(p1_hbm.at[pl.ds(0, CHUNK)], g2, gs).wait()

            def add_row(rw, c):
                for j in range(D // 16):
                    s2 = pl.ds(j * 16, 16)
                    g1[rw, s2] = g1[rw, s2] + g2[rw, s2]
                return c

            lax.fori_loop(0, CHUNK, add_row, 0)
            dst = pl.multiple_of(e0 + bi * CHUNK, 8)
            pltpu.async_copy(g1, g_hbm.at[pl.ds(dst, CHUNK)], ws)
        return carry

    lax.fori_loop(0, nb // NSLOT, outer, 0)

    for k in range(NSLOT):
        pltpu.make_async_copy(
            slots[k][0], g_hbm.at[pl.ds(0, CHUNK)], slots[k][3]).wait()


def _gather_sum(p1, p2, sidx, ridx):
    mesh = plsc.VectorSubcoreMesh(core_axis_name="c", subcore_axis_name="s",
                                  num_cores=NC, num_subcores=NS)
    kern = pl.kernel(
        _gather_sum_body,
        out_type=jax.ShapeDtypeStruct((N_PAD, D), jnp.float32),
        mesh=mesh,
        scratch_types=(
            [pltpu.VMEM((PER_W_MAX,), jnp.int32)] * 2
            + [pltpu.VMEM((CHUNK, D), jnp.float32)] * (2 * NSLOT)
            + [pltpu.SemaphoreType.DMA] * (2 * NSLOT)
        ),
    )
    return kern(p1, p2, sidx, ridx)


def _epilogue_body(g_ref, e_ref, w3_ref, b_ref, o_ref):
    o_ref[...] = (g_ref[...]
                  + jnp.dot(e_ref[...], w3_ref[...],
                            preferred_element_type=jnp.float32)
                  + b_ref[...])


def _epilogue(g, edge_attr, w3, b2d):
    blk = 2000
    grid = N_EDGES // blk
    return pl.pallas_call(
        _epilogue_body,
        grid=(grid,),
        in_specs=[
            pl.BlockSpec((blk, D), lambda i: (i, 0)),
            pl.BlockSpec((blk, D_EDGE), lambda i: (i, 0)),
            pl.BlockSpec((D_EDGE, D), lambda i: (0, 0)),
            pl.BlockSpec((1, D), lambda i: (0, 0)),
        ],
        out_specs=pl.BlockSpec((blk, D), lambda i: (i, 0)),
        out_shape=jax.ShapeDtypeStruct((N_EDGES, D), jnp.float32),
    )(g, edge_attr, w3, b2d)


def kernel(node_attr, edge_index, edge_attr, edge_world_index, edge_world_attr, W, b):
    w1 = W[:D]
    w2 = W[D:2 * D]
    w3 = W[2 * D:]
    b2d = b.reshape(1, D)

    p1, p2 = _project(node_attr, w1, w2)

    pad = IDX_PAD - N_EDGES
    sidx = jnp.pad(edge_index[0], (0, pad))
    ridx = jnp.pad(edge_index[1], (0, pad))

    g = _gather_sum(p1, p2, sidx, ridx)

    edge_attr_ = _epilogue(g, edge_attr, w3, b2d)
    return (node_attr, edge_attr_, edge_index, edge_world_index, edge_world_attr)


# speedup vs baseline: 1.0286x; 1.0286x over previous
import functools

import jax
import jax.numpy as jnp
from jax import lax
from jax.experimental import pallas as pl
from jax.experimental.pallas import tpu as pltpu
from jax.experimental.pallas import tpu_sc as plsc

N_NODES = 10000
N_EDGES = 320000
D = 128
D_EDGE = 16

NC, NS = 2, 16
NW = NC * NS
CHUNK = 160
NSLOT = 2
BATCH_A = 64
BATCH_B = 64
BATCHES = BATCH_A + BATCH_B
PER_W_MAX = CHUNK * BATCH_A
N_PAD = CHUNK * BATCHES * NS
IDX_PAD = N_PAD + (BATCH_A - BATCH_B) * CHUNK


def _project_body(x_ref, w1_ref, w2_ref, p1_ref, p2_ref):
    x = x_ref[...]
    p1_ref[...] = jnp.dot(x, w1_ref[...], preferred_element_type=jnp.float32)
    p2_ref[...] = jnp.dot(x, w2_ref[...], preferred_element_type=jnp.float32)


def _project(node_attr, w1, w2):
    blk = 1000
    grid = N_NODES // blk
    return pl.pallas_call(
        _project_body,
        grid=(grid,),
        in_specs=[
            pl.BlockSpec((blk, D), lambda i: (i, 0)),
            pl.BlockSpec((D, D), lambda i: (0, 0)),
            pl.BlockSpec((D, D), lambda i: (0, 0)),
        ],
        out_specs=[
            pl.BlockSpec((blk, D), lambda i: (i, 0)),
            pl.BlockSpec((blk, D), lambda i: (i, 0)),
        ],
        out_shape=[
            jax.ShapeDtypeStruct((N_NODES, D), jnp.float32),
            jax.ShapeDtypeStruct((N_NODES, D), jnp.float32),
        ],
    )(node_attr, w1, w2)


def _gather_sum_body(p1_hbm, p2_hbm, sidx_hbm, ridx_hbm, g_hbm,
                     idxs_v, idxr_v,
                     b1_0, b2_0, b1_1, b2_1, gs0, gs1, ws0, ws1):
    cid = lax.axis_index("c")
    sid = lax.axis_index("s")
    nb = jnp.where(cid == 0, BATCH_A, BATCH_B)
    e0 = jnp.where(cid == 0,
                   sid * (BATCH_A * CHUNK),
                   NS * (BATCH_A * CHUNK) + sid * (BATCH_B * CHUNK))
    e0 = pl.multiple_of(e0, 8)
    pltpu.sync_copy(sidx_hbm.at[pl.ds(e0, PER_W_MAX)], idxs_v)
    pltpu.sync_copy(ridx_hbm.at[pl.ds(e0, PER_W_MAX)], idxr_v)

    slots = ((b1_0, b2_0, gs0, ws0), (b1_1, b2_1, gs1, ws1))

    def issue_gather(b, g1, g2, gs):
        off = pl.multiple_of(b * CHUNK, 8)
        pltpu.async_copy(p1_hbm.at[idxs_v.at[pl.ds(off, CHUNK)]], g1, gs)
        pltpu.async_copy(p2_hbm.at[idxr_v.at[pl.ds(off, CHUNK)]], g2, gs)

    for k in range(NSLOT):
        issue_gather(k, slots[k][0], slots[k][1], slots[k][2])

    def outer(it, carry):
        for k in range(NSLOT):
            g1, g2, gs, ws = slots[k]
            kp = (k + NSLOT - 1) % NSLOT
            g1p, g2p, gsp, wsp = slots[kp]
            bi = it * NSLOT + k

            @pl.when(jnp.logical_and(bi >= 1, bi + NSLOT - 1 < nb))
            def _():
                pltpu.make_async_copy(
                    g1p, g_hbm.at[pl.ds(0, CHUNK)], wsp).wait()
                issue_gather(bi + NSLOT - 1, g1p, g2p, gsp)

            pltpu.make_async_copy(p1_hbm.at[pl.ds(0, CHUNK)], g1, gs).wait()
            pltpu.make_async_copy(p1_hbm.at[pl.ds(0, CHUNK)], g2, gs).wait()

            def add_row(rw, c):
                for j in range(D // 16):
                    s2 = pl.ds(j * 16, 16)
                    g1[rw, s2] = g1[rw, s2] + g2[rw, s2]
                return c

            lax.fori_loop(0, CHUNK, add_row, 0)
            dst = pl.multiple_of(e0 + bi * CHUNK, 8)
            pltpu.async_copy(g1, g_hbm.at[pl.ds(dst, CHUNK)], ws)
        return carry

    lax.fori_loop(0, nb // NSLOT, outer, 0)

    for k in range(NSLOT):
        pltpu.make_async_copy(
            slots[k][0], g_hbm.at[pl.ds(0, CHUNK)], slots[k][3]).wait()


def _gather_sum(p1, p2, sidx, ridx):
    mesh = plsc.VectorSubcoreMesh(core_axis_name="c", subcore_axis_name="s",
                                  num_cores=NC, num_subcores=NS)
    kern = pl.kernel(
        _gather_sum_body,
        out_type=jax.ShapeDtypeStruct((N_PAD, D), jnp.float32),
        mesh=mesh,
        scratch_types=(
            [pltpu.VMEM((PER_W_MAX,), jnp.int32)] * 2
            + [pltpu.VMEM((CHUNK, D), jnp.float32)] * (2 * NSLOT)
            + [pltpu.SemaphoreType.DMA] * (2 * NSLOT)
        ),
    )
    return kern(p1, p2, sidx, ridx)


def _epilogue_body(g_ref, e_ref, w3_ref, b_ref, o_ref):
    o_ref[...] = (g_ref[...]
                  + jnp.dot(e_ref[...], w3_ref[...],
                            preferred_element_type=jnp.float32)
                  + b_ref[...])


def _epilogue(g, edge_attr, w3, b2d):
    blk = 4000
    grid = N_EDGES // blk
    return pl.pallas_call(
        _epilogue_body,
        grid=(grid,),
        in_specs=[
            pl.BlockSpec((blk, D), lambda i: (i, 0)),
            pl.BlockSpec((blk, D_EDGE), lambda i: (i, 0)),
            pl.BlockSpec((D_EDGE, D), lambda i: (0, 0)),
            pl.BlockSpec((1, D), lambda i: (0, 0)),
        ],
        out_specs=pl.BlockSpec((blk, D), lambda i: (i, 0)),
        out_shape=jax.ShapeDtypeStruct((N_EDGES, D), jnp.float32),
    )(g, edge_attr, w3, b2d)


def kernel(node_attr, edge_index, edge_attr, edge_world_index, edge_world_attr, W, b):
    w1 = W[:D]
    w2 = W[D:2 * D]
    w3 = W[2 * D:]
    b2d = b.reshape(1, D)

    p1, p2 = _project(node_attr, w1, w2)

    pad = IDX_PAD - N_EDGES
    sidx = jnp.pad(edge_index[0], (0, pad))
    ridx = jnp.pad(edge_index[1], (0, pad))

    g = _gather_sum(p1, p2, sidx, ridx)

    edge_attr_ = _epilogue(g, edge_attr, w3, b2d)
    return (node_attr, edge_attr_, edge_index, edge_world_index, edge_world_attr)


# speedup vs baseline: 1.0452x; 1.0161x over previous
import functools

import jax
import jax.numpy as jnp
from jax import lax
from jax.experimental import pallas as pl
from jax.experimental.pallas import tpu as pltpu
from jax.experimental.pallas import tpu_sc as plsc

N_NODES = 10000
N_EDGES = 320000
D = 128
D_EDGE = 16

NC, NS = 2, 16
NW = NC * NS
CHUNK = 160
NSLOT = 2
BATCH_A = 64
BATCH_B = 64
BATCHES = BATCH_A + BATCH_B
PER_W_MAX = CHUNK * BATCH_A
N_PAD = CHUNK * BATCHES * NS
IDX_PAD = N_PAD + (BATCH_A - BATCH_B) * CHUNK


def _project_body(x_ref, w1_ref, w2_ref, p1_ref, p2_ref):
    x = x_ref[...]
    p1_ref[...] = jnp.dot(x, w1_ref[...], preferred_element_type=jnp.float32)
    p2_ref[...] = jnp.dot(x, w2_ref[...], preferred_element_type=jnp.float32)


def _project(node_attr, w1, w2):
    blk = 1000
    grid = N_NODES // blk
    return pl.pallas_call(
        _project_body,
        grid=(grid,),
        in_specs=[
            pl.BlockSpec((blk, D), lambda i: (i, 0)),
            pl.BlockSpec((D, D), lambda i: (0, 0)),
            pl.BlockSpec((D, D), lambda i: (0, 0)),
        ],
        out_specs=[
            pl.BlockSpec((blk, D), lambda i: (i, 0)),
            pl.BlockSpec((blk, D), lambda i: (i, 0)),
        ],
        out_shape=[
            jax.ShapeDtypeStruct((N_NODES, D), jnp.float32),
            jax.ShapeDtypeStruct((N_NODES, D), jnp.float32),
        ],
    )(node_attr, w1, w2)


def _gather_sum_body(p1_hbm, p2_hbm, sidx_hbm, ridx_hbm, g_hbm,
                     idxs_v, idxr_v,
                     b1_0, b2_0, b1_1, b2_1, gs0, gs1, ws0, ws1):
    cid = lax.axis_index("c")
    sid = lax.axis_index("s")
    nb = jnp.where(cid == 0, BATCH_A, BATCH_B)
    e0 = jnp.where(cid == 0,
                   sid * (BATCH_A * CHUNK),
                   NS * (BATCH_A * CHUNK) + sid * (BATCH_B * CHUNK))
    e0 = pl.multiple_of(e0, 8)
    pltpu.sync_copy(sidx_hbm.at[pl.ds(e0, PER_W_MAX)], idxs_v)
    pltpu.sync_copy(ridx_hbm.at[pl.ds(e0, PER_W_MAX)], idxr_v)

    slots = ((b1_0, b2_0, gs0, ws0), (b1_1, b2_1, gs1, ws1))

    def issue_gather(b, g1, g2, gs):
        off = pl.multiple_of(b * CHUNK, 8)
        pltpu.async_copy(p1_hbm.at[idxs_v.at[pl.ds(off, CHUNK)]], g1, gs)
        pltpu.async_copy(p2_hbm.at[idxr_v.at[pl.ds(off, CHUNK)]], g2, gs)

    for k in range(NSLOT):
        issue_gather(k, slots[k][0], slots[k][1], slots[k][2])

    def outer(it, carry):
        for k in range(NSLOT):
            g1, g2, gs, ws = slots[k]
            kp = (k + NSLOT - 1) % NSLOT
            g1p, g2p, gsp, wsp = slots[kp]
            bi = it * NSLOT + k

            @pl.when(jnp.logical_and(bi >= 1, bi + NSLOT - 1 < nb))
            def _():
                pltpu.make_async_copy(
                    g1p, g_hbm.at[pl.ds(0, CHUNK)], wsp).wait()
                issue_gather(bi + NSLOT - 1, g1p, g2p, gsp)

            pltpu.make_async_copy(p1_hbm.at[pl.ds(0, CHUNK)], g1, gs).wait()
            pltpu.make_async_copy(p1_hbm.at[pl.ds(0, CHUNK)], g2, gs).wait()

            def add_row(rw, c):
                for j in range(D // 16):
                    s2 = pl.ds(j * 16, 16)
                    g1[rw, s2] = g1[rw, s2] + g2[rw, s2]
                return c

            lax.fori_loop(0, CHUNK, add_row, 0)
            dst = pl.multiple_of(e0 + bi * CHUNK, 8)
            pltpu.async_copy(g1, g_hbm.at[pl.ds(dst, CHUNK)], ws)
        return carry

    lax.fori_loop(0, nb // NSLOT, outer, 0)

    for k in range(NSLOT):
        pltpu.make_async_copy(
            slots[k][0], g_hbm.at[pl.ds(0, CHUNK)], slots[k][3]).wait()


def _gather_sum(p1, p2, sidx, ridx):
    mesh = plsc.VectorSubcoreMesh(core_axis_name="c", subcore_axis_name="s",
                                  num_cores=NC, num_subcores=NS)
    kern = pl.kernel(
        _gather_sum_body,
        out_type=jax.ShapeDtypeStruct((N_PAD, D), jnp.float32),
        mesh=mesh,
        scratch_types=(
            [pltpu.VMEM((PER_W_MAX,), jnp.int32)] * 2
            + [pltpu.VMEM((CHUNK, D), jnp.float32)] * (2 * NSLOT)
            + [pltpu.SemaphoreType.DMA] * (2 * NSLOT)
        ),
    )
    return kern(p1, p2, sidx, ridx)


def _epilogue_body(g_ref, e_ref, w3_ref, b_ref, o_ref):
    o_ref[...] = (g_ref[...]
                  + jnp.dot(e_ref[...], w3_ref[...],
                            preferred_element_type=jnp.float32)
                  + b_ref[...])


def _epilogue(g, edge_attr, w3, b2d):
    blk = 8000
    grid = N_EDGES // blk
    return pl.pallas_call(
        _epilogue_body,
        grid=(grid,),
        in_specs=[
            pl.BlockSpec((blk, D), lambda i: (i, 0)),
            pl.BlockSpec((blk, D_EDGE), lambda i: (i, 0)),
            pl.BlockSpec((D_EDGE, D), lambda i: (0, 0)),
            pl.BlockSpec((1, D), lambda i: (0, 0)),
        ],
        out_specs=pl.BlockSpec((blk, D), lambda i: (i, 0)),
        out_shape=jax.ShapeDtypeStruct((N_EDGES, D), jnp.float32),
    )(g, edge_attr, w3, b2d)


def kernel(node_attr, edge_index, edge_attr, edge_world_index, edge_world_attr, W, b):
    w1 = W[:D]
    w2 = W[D:2 * D]
    w3 = W[2 * D:]
    b2d = b.reshape(1, D)

    p1, p2 = _project(node_attr, w1, w2)

    pad = IDX_PAD - N_EDGES
    sidx = jnp.pad(edge_index[0], (0, pad))
    ridx = jnp.pad(edge_index[1], (0, pad))

    g = _gather_sum(p1, p2, sidx, ridx)

    edge_attr_ = _epilogue(g, edge_attr, w3, b2d)
    return (node_attr, edge_attr_, edge_index, edge_world_index, edge_world_attr)


# speedup vs baseline: 1.0626x; 1.0166x over previous
import functools

import jax
import jax.numpy as jnp
from jax import lax
from jax.experimental import pallas as pl
from jax.experimental.pallas import tpu as pltpu
from jax.experimental.pallas import tpu_sc as plsc

N_NODES = 10000
N_EDGES = 320000
D = 128
D_EDGE = 16

NC, NS = 2, 16
NW = NC * NS
CHUNK = 160
NSLOT = 2
BATCH_A = 64
BATCH_B = 64
BATCHES = BATCH_A + BATCH_B
PER_W_MAX = CHUNK * BATCH_A
N_PAD = CHUNK * BATCHES * NS
IDX_PAD = N_PAD + (BATCH_A - BATCH_B) * CHUNK


def _project_body(x_ref, w1_ref, w2_ref, p1_ref, p2_ref):
    x = x_ref[...]
    p1_ref[...] = jnp.dot(x, w1_ref[...], preferred_element_type=jnp.float32)
    p2_ref[...] = jnp.dot(x, w2_ref[...], preferred_element_type=jnp.float32)


def _project(node_attr, w1, w2):
    blk = 1000
    grid = N_NODES // blk
    return pl.pallas_call(
        _project_body,
        grid=(grid,),
        in_specs=[
            pl.BlockSpec((blk, D), lambda i: (i, 0)),
            pl.BlockSpec((D, D), lambda i: (0, 0)),
            pl.BlockSpec((D, D), lambda i: (0, 0)),
        ],
        out_specs=[
            pl.BlockSpec((blk, D), lambda i: (i, 0)),
            pl.BlockSpec((blk, D), lambda i: (i, 0)),
        ],
        out_shape=[
            jax.ShapeDtypeStruct((N_NODES, D), jnp.float32),
            jax.ShapeDtypeStruct((N_NODES, D), jnp.float32),
        ],
    )(node_attr, w1, w2)


def _gather_sum_body(p1_hbm, p2_hbm, sidx_hbm, ridx_hbm, g_hbm,
                     idxs_v, idxr_v,
                     b1_0, b2_0, b1_1, b2_1, gs0, gs1, ws0, ws1):
    cid = lax.axis_index("c")
    sid = lax.axis_index("s")
    nb = jnp.where(cid == 0, BATCH_A, BATCH_B)
    e0 = jnp.where(cid == 0,
                   sid * (BATCH_A * CHUNK),
                   NS * (BATCH_A * CHUNK) + sid * (BATCH_B * CHUNK))
    e0 = pl.multiple_of(e0, 8)
    pltpu.sync_copy(sidx_hbm.at[pl.ds(e0, PER_W_MAX)], idxs_v)
    pltpu.sync_copy(ridx_hbm.at[pl.ds(e0, PER_W_MAX)], idxr_v)

    slots = ((b1_0, b2_0, gs0, ws0), (b1_1, b2_1, gs1, ws1))

    def issue_gather(b, g1, g2, gs):
        off = pl.multiple_of(b * CHUNK, 8)
        pltpu.async_copy(p1_hbm.at[idxs_v.at[pl.ds(off, CHUNK)]], g1, gs)
        pltpu.async_copy(p2_hbm.at[idxr_v.at[pl.ds(off, CHUNK)]], g2, gs)

    for k in range(NSLOT):
        issue_gather(k, slots[k][0], slots[k][1], slots[k][2])

    def outer(it, carry):
        for k in range(NSLOT):
            g1, g2, gs, ws = slots[k]
            kp = (k + NSLOT - 1) % NSLOT
            g1p, g2p, gsp, wsp = slots[kp]
            bi = it * NSLOT + k

            @pl.when(jnp.logical_and(bi >= 1, bi + NSLOT - 1 < nb))
            def _():
                pltpu.make_async_copy(
                    g1p, g_hbm.at[pl.ds(0, CHUNK)], wsp).wait()
                issue_gather(bi + NSLOT - 1, g1p, g2p, gsp)

            pltpu.make_async_copy(p1_hbm.at[pl.ds(0, CHUNK)], g1, gs).wait()
            pltpu.make_async_copy(p1_hbm.at[pl.ds(0, CHUNK)], g2, gs).wait()

            def add_row(rw, c):
                for j in range(D // 16):
                    s2 = pl.ds(j * 16, 16)
                    g1[rw, s2] = g1[rw, s2] + g2[rw, s2]
                return c

            lax.fori_loop(0, CHUNK, add_row, 0)
            dst = pl.multiple_of(e0 + bi * CHUNK, 8)
            pltpu.async_copy(g1, g_hbm.at[pl.ds(dst, CHUNK)], ws)
        return carry

    lax.fori_loop(0, nb // NSLOT, outer, 0)

    for k in range(NSLOT):
        pltpu.make_async_copy(
            slots[k][0], g_hbm.at[pl.ds(0, CHUNK)], slots[k][3]).wait()


def _gather_sum(p1, p2, sidx, ridx):
    mesh = plsc.VectorSubcoreMesh(core_axis_name="c", subcore_axis_name="s",
                                  num_cores=NC, num_subcores=NS)
    kern = pl.kernel(
        _gather_sum_body,
        out_type=jax.ShapeDtypeStruct((N_PAD, D), jnp.float32),
        mesh=mesh,
        scratch_types=(
            [pltpu.VMEM((PER_W_MAX,), jnp.int32)] * 2
            + [pltpu.VMEM((CHUNK, D), jnp.float32)] * (2 * NSLOT)
            + [pltpu.SemaphoreType.DMA] * (2 * NSLOT)
        ),
    )
    return kern(p1, p2, sidx, ridx)


def _epilogue_body(g_ref, e_ref, w3_ref, b_ref, o_ref):
    o_ref[...] = (g_ref[...]
                  + jnp.dot(e_ref[...], w3_ref[...],
                            preferred_element_type=jnp.float32)
                  + b_ref[...])


def _epilogue(g, edge_attr, w3, b2d):
    blk = 16000
    grid = N_EDGES // blk
    return pl.pallas_call(
        _epilogue_body,
        grid=(grid,),
        in_specs=[
            pl.BlockSpec((blk, D), lambda i: (i, 0)),
            pl.BlockSpec((blk, D_EDGE), lambda i: (i, 0)),
            pl.BlockSpec((D_EDGE, D), lambda i: (0, 0)),
            pl.BlockSpec((1, D), lambda i: (0, 0)),
        ],
        out_specs=pl.BlockSpec((blk, D), lambda i: (i, 0)),
        out_shape=jax.ShapeDtypeStruct((N_EDGES, D), jnp.float32),
    )(g, edge_attr, w3, b2d)


def kernel(node_attr, edge_index, edge_attr, edge_world_index, edge_world_attr, W, b):
    w1 = W[:D]
    w2 = W[D:2 * D]
    w3 = W[2 * D:]
    b2d = b.reshape(1, D)

    p1, p2 = _project(node_attr, w1, w2)

    pad = IDX_PAD - N_EDGES
    sidx = jnp.pad(edge_index[0], (0, pad))
    ridx = jnp.pad(edge_index[1], (0, pad))

    g = _gather_sum(p1, p2, sidx, ridx)

    edge_attr_ = _epilogue(g, edge_attr, w3, b2d)
    return (node_attr, edge_attr_, edge_index, edge_world_index, edge_world_attr)
